# baseline (speedup 1.0000x reference)
import jax
import jax.numpy as jnp
from jax import lax
from jax.experimental import pallas as pl
from jax.experimental.pallas import tpu as pltpu
from jax.experimental.pallas import tpu_sc as plsc

B, S, H = 4, 2048, 768
N = B * S
NW = 32
TPW = N // NW
CHUNK = 64
NCHUNK = TPW // CHUNK
LN_EPS = 1e-12

BLK = 1024
NBLK_B = S // BLK
NBLK = N // BLK


def _sc_body(ids_hbm, word_hbm, annidx_hbm, anntab_hbm,
             wraw_out, ann_out,
             iw0, iw1, w0, w1, ai, abuf, gs0, gs1, ws0, ws1, asem,
             *, tpw, do_ann):
    nchunk = tpw // CHUNK
    c = lax.axis_index("c")
    s = lax.axis_index("s")
    wid = c * 16 + s
    base = wid * tpw

    def fire_gather(chunk, iw, wbuf, gs):
        t0 = base + chunk * CHUNK
        pltpu.sync_copy(ids_hbm.at[pl.ds(t0, CHUNK)], iw)
        pltpu.async_copy(word_hbm.at[iw], wbuf, gs)

    def wait_gather(iw, wbuf, gs):
        pltpu.make_async_copy(word_hbm.at[iw], wbuf, gs).wait()

    def fire_write(chunk, wbuf, ws):
        t0 = base + chunk * CHUNK
        pltpu.async_copy(wbuf, wraw_out.at[pl.ds(t0, CHUNK), :], ws)

    def wait_write(wbuf, ws):
        pltpu.make_async_copy(wbuf, wraw_out.at[pl.ds(base, CHUNK), :],
                              ws).wait()

    if do_ann:
        @pl.when(wid == 0)
        def _():
            pltpu.sync_copy(annidx_hbm, ai)
            pltpu.async_copy(anntab_hbm.at[ai], abuf, asem)

    fire_gather(0, iw0, w0, gs0)

    def body(j, _):
        a = 2 * j

        @pl.when(j > 0)
        def _():
            wait_write(w1, ws1)

        fire_gather(a + 1, iw1, w1, gs1)
        wait_gather(iw0, w0, gs0)
        fire_write(a, w0, ws0)
        wait_gather(iw1, w1, gs1)
        wait_write(w0, ws0)

        @pl.when(j < nchunk // 2 - 1)
        def _():
            fire_gather(a + 2, iw0, w0, gs0)

        fire_write(a + 1, w1, ws1)
        return 0

    lax.fori_loop(0, nchunk // 2, body, 0)
    wait_write(w1, ws1)

    if do_ann:
        @pl.when(wid == 0)
        def _():
            pltpu.make_async_copy(anntab_hbm.at[ai], abuf, asem).wait()
            pltpu.sync_copy(abuf, ann_out)


def _sc_gather(ids, ann_idx_pad, word_emb, ann_table, do_ann):
    import functools
    nrows = ids.shape[0]
    mesh = plsc.VectorSubcoreMesh(core_axis_name="c", subcore_axis_name="s",
                                  num_cores=2, num_subcores=16)
    fn = pl.kernel(
        functools.partial(_sc_body, tpw=nrows // NW, do_ann=do_ann),
        out_type=[
            jax.ShapeDtypeStruct((nrows, H), jnp.float32),
            jax.ShapeDtypeStruct((16, H), jnp.float32),
        ],
        mesh=mesh,
        scratch_types=[
            pltpu.VMEM((CHUNK,), jnp.int32),
            pltpu.VMEM((CHUNK,), jnp.int32),
            pltpu.VMEM((CHUNK, H), jnp.float32),
            pltpu.VMEM((CHUNK, H), jnp.float32),
            pltpu.VMEM((16,), jnp.int32),
            pltpu.VMEM((16, H), jnp.float32),
            pltpu.SemaphoreType.DMA,
            pltpu.SemaphoreType.DMA,
            pltpu.SemaphoreType.DMA,
            pltpu.SemaphoreType.DMA,
            pltpu.SemaphoreType.DMA,
        ],
    )
    return fn(ids, word_emb, ann_idx_pad, ann_table)


def _ln(y, g, b):
    mu = jnp.mean(y, axis=1, keepdims=True)
    d = y - mu
    var = jnp.mean(d * d, axis=1, keepdims=True)
    return d * lax.rsqrt(var + LN_EPS) * g + b


def _tc_ln_body(nbat, nblk_l, wraw_ref, pos_ref, ttid_ref, tt_ref, g_ref,
                b_ref, *rest):
    sent_ref, emb_ref, part_ref = rest[-3:]
    i = pl.program_id(0)
    r = lax.rem(i, nbat) * NBLK_B + i // nbat
    tt0 = tt_ref[pl.ds(0, 1), :]
    tt1 = tt_ref[pl.ds(1, 1), :]
    tts = ttid_ref[pl.ds(r, 1), :]
    is1 = (jnp.transpose(tts, (1, 0)) == 1)
    x = wraw_ref[...] + pos_ref[...] + jnp.where(is1, tt1, tt0)
    sent_ref[...] = x
    bsum = jnp.sum(x, axis=0, keepdims=True)
    row0 = (lax.broadcasted_iota(jnp.int32, (8, 1), 0) == 0)
    part_ref[...] = jnp.where(row0, bsum, 0.0)
    emb_ref[...] = _ln(x, g_ref[...], b_ref[...])


def _tc_ln(wraw, pos_emb, tt_ids, tt_emb, gamma, beta, base_blk, aliases):
    import functools
    nrows = wraw.shape[0]
    nbat = nrows // S
    nblk_l = nrows // BLK
    loc = lambda i: (lax.rem(i, nbat) * NBLK_B + i // nbat, 0)
    glb = lambda i: (base_blk + lax.rem(i, nbat) * NBLK_B + i // nbat, 0)
    in_specs = [
        pl.BlockSpec((BLK, H), loc),
        pl.BlockSpec((BLK, H), lambda i: (i // nbat, 0)),
        pl.BlockSpec((nblk_l, BLK), lambda i: (0, 0)),
        pl.BlockSpec((2, H), lambda i: (0, 0)),
        pl.BlockSpec((1, H), lambda i: (0, 0)),
        pl.BlockSpec((1, H), lambda i: (0, 0)),
    ]
    args = [wraw, pos_emb, tt_ids, tt_emb, gamma, beta]
    io_alias = {}
    if aliases is not None:
        in_specs += [pl.BlockSpec((8, H), lambda i: (0, 0)),
                     pl.BlockSpec((8, H), lambda i: (0, 0))]
        args += [aliases[0], aliases[1]]
        io_alias = {6: 0, 7: 1}
    return pl.pallas_call(
        functools.partial(_tc_ln_body, nbat, nblk_l),
        grid=(nbat * NBLK_B,),
        in_specs=in_specs,
        out_specs=[
            pl.BlockSpec((BLK, H), glb),
            pl.BlockSpec((BLK, H), glb),
            pl.BlockSpec((8, H), loc),
        ],
        out_shape=[
            jax.ShapeDtypeStruct((N, H), jnp.float32),
            jax.ShapeDtypeStruct((N, H), jnp.float32),
            jax.ShapeDtypeStruct((nblk_l * 8, H), jnp.float32),
        ],
        input_output_aliases=io_alias,
    )(*args)


def _tc_patch_body(emb_in_ref, sent_ref, part_ref, ann_ref, sw_ref, aw_ref,
                   g_ref, b_ref, emb_ref, annout_ref, ann_emb_s):
    b = pl.program_id(0)

    @pl.when(b == 0)
    def _():
        p = part_ref[...]
        io = lax.broadcasted_iota(jnp.int32, (8, NBLK * 8), 0)
        ii = lax.broadcasted_iota(jnp.int32, (8, NBLK * 8), 1)
        sel = (ii // (8 * NBLK_B) == io).astype(jnp.float32)
        m = lax.dot_general(sel, p, (((1,), (0,)), ((), ())),
                            precision=lax.Precision.HIGHEST,
                            preferred_element_type=jnp.float32) * (1.0 / S)
        ann = ann_ref[...]
        u = lax.dot_general(m, sw_ref[...], (((1,), (1,)), ((), ())),
                            precision=lax.Precision.HIGHEST,
                            preferred_element_type=jnp.float32)
        v = lax.dot_general(ann, aw_ref[...], (((1,), (1,)), ((), ())),
                            precision=lax.Precision.HIGHEST,
                            preferred_element_type=jnp.float32)
        alpha = jnp.sum(u * v, axis=1, keepdims=True)
        ann_emb = alpha * ann
        ann_emb_s[...] = ann_emb
        annout_ref[...] = lax.slice(ann_emb, (0, 0), (B, H))

    ann_emb_b = ann_emb_s[pl.ds(b, 1), :]
    y0 = _ln(sent_ref[pl.ds(0, 1), :] + ann_emb_b, g_ref[...], b_ref[...])
    row0 = (lax.broadcasted_iota(jnp.int32, (8, 1), 0) == 0)
    emb_ref[...] = jnp.where(row0, y0, emb_in_ref[...])


def _tc_alpha_patch(emb0, sent, partials, ann_rows, sent_W, annotator_W,
                    gamma, beta):
    return pl.pallas_call(
        _tc_patch_body,
        grid=(B,),
        in_specs=[
            pl.BlockSpec((8, H), lambda b: (b * (S // 8), 0)),
            pl.BlockSpec((8, H), lambda b: (b * (S // 8), 0)),
            pl.BlockSpec((NBLK * 8, H), lambda b: (0, 0)),
            pl.BlockSpec((8, H), lambda b: (0, 0)),
            pl.BlockSpec((H, H), lambda b: (0, 0)),
            pl.BlockSpec((H, H), lambda b: (0, 0)),
            pl.BlockSpec((1, H), lambda b: (0, 0)),
            pl.BlockSpec((1, H), lambda b: (0, 0)),
        ],
        out_specs=[
            pl.BlockSpec((8, H), lambda b: (b * (S // 8), 0)),
            pl.BlockSpec((B, H), lambda b: (0, 0)),
        ],
        out_shape=[
            jax.ShapeDtypeStruct((N, H), jnp.float32),
            jax.ShapeDtypeStruct((B, H), jnp.float32),
        ],
        input_output_aliases={0: 0},
        scratch_shapes=[pltpu.VMEM((8, H), jnp.float32)],
    )(emb0, sent, partials, ann_rows, sent_W, annotator_W, gamma, beta)


def kernel(input_ids, token_type_ids, annotator_ids, word_emb, tt_emb,
           pos_emb, sent_W, annotator_W, ann_table, ln_gamma, ln_beta):
    ids = input_ids.reshape(-1).astype(jnp.int32)
    tt_ids = token_type_ids.reshape(NBLK, BLK).astype(jnp.int32)
    ann_idx_pad = jnp.tile(annotator_ids.astype(jnp.int32), 16 // B)
    g1 = ln_gamma.reshape(1, H)
    b1 = ln_beta.reshape(1, H)
    half = N // 2

    wraw1, ann_rows = _sc_gather(ids[:half], ann_idx_pad, word_emb,
                                 ann_table, True)
    wraw2, _ = _sc_gather(ids[half:], ann_idx_pad, word_emb, ann_table,
                          False)
    sent0, embh, p1 = _tc_ln(wraw1, pos_emb, tt_ids[:NBLK // 2], tt_emb,
                             g1, b1, 0, None)
    sent, emb0, p2 = _tc_ln(wraw2, pos_emb, tt_ids[NBLK // 2:], tt_emb,
                            g1, b1, NBLK // 2, (sent0, embh))
    partials = jnp.concatenate([p1, p2], axis=0)
    emb, ann_emb = _tc_alpha_patch(emb0, sent, partials, ann_rows, sent_W,
                                   annotator_W, g1, b1)
    return (emb.reshape(B, S, H), ann_emb, sent.reshape(B, S, H))


# speedup vs baseline: 1.0522x; 1.0522x over previous
import jax
import jax.numpy as jnp
from jax import lax
from jax.experimental import pallas as pl
from jax.experimental.pallas import tpu as pltpu
from jax.experimental.pallas import tpu_sc as plsc

B, S, H = 4, 2048, 768
N = B * S
NW = 32
TPW = N // NW
CHUNK = 64
NCHUNK = TPW // CHUNK
LN_EPS = 1e-12

BLK = 1024
NBLK_B = S // BLK
NBLK = N // BLK


def _sc_body(ids_hbm, word_hbm, annidx_hbm, anntab_hbm,
             wraw_out, ann_out,
             iw0, iw1, w0, w1, ai, abuf, gs0, gs1, ws0, ws1, asem):
    c = lax.axis_index("c")
    s = lax.axis_index("s")
    wid = c * 16 + s
    base = wid * TPW

    def fire_gather(chunk, iw, wbuf, gs):
        t0 = base + chunk * CHUNK
        pltpu.sync_copy(ids_hbm.at[pl.ds(t0, CHUNK)], iw)
        pltpu.async_copy(word_hbm.at[iw], wbuf, gs)

    def wait_gather(iw, wbuf, gs):
        pltpu.make_async_copy(word_hbm.at[iw], wbuf, gs).wait()

    def fire_write(chunk, wbuf, ws):
        t0 = base + chunk * CHUNK
        pltpu.async_copy(wbuf, wraw_out.at[pl.ds(t0, CHUNK), :], ws)

    def wait_write(wbuf, ws):
        pltpu.make_async_copy(wbuf, wraw_out.at[pl.ds(base, CHUNK), :],
                              ws).wait()

    @pl.when(wid == 0)
    def _():
        pltpu.sync_copy(annidx_hbm, ai)
        pltpu.async_copy(anntab_hbm.at[ai], abuf, asem)

    fire_gather(0, iw0, w0, gs0)

    def body(j, _):
        a = 2 * j

        @pl.when(j > 0)
        def _():
            wait_write(w1, ws1)

        fire_gather(a + 1, iw1, w1, gs1)
        wait_gather(iw0, w0, gs0)
        fire_write(a, w0, ws0)
        wait_gather(iw1, w1, gs1)
        wait_write(w0, ws0)

        @pl.when(j < NCHUNK // 2 - 1)
        def _():
            fire_gather(a + 2, iw0, w0, gs0)

        fire_write(a + 1, w1, ws1)
        return 0

    lax.fori_loop(0, NCHUNK // 2, body, 0)
    wait_write(w1, ws1)

    @pl.when(wid == 0)
    def _():
        pltpu.make_async_copy(anntab_hbm.at[ai], abuf, asem).wait()
        pltpu.sync_copy(abuf, ann_out)


def _sc_gather(ids, ann_idx_pad, word_emb, ann_table):
    mesh = plsc.VectorSubcoreMesh(core_axis_name="c", subcore_axis_name="s",
                                  num_cores=2, num_subcores=16)
    fn = pl.kernel(
        _sc_body,
        out_type=[
            jax.ShapeDtypeStruct((N, H), jnp.float32),
            jax.ShapeDtypeStruct((16, H), jnp.float32),
        ],
        mesh=mesh,
        scratch_types=[
            pltpu.VMEM((CHUNK,), jnp.int32),
            pltpu.VMEM((CHUNK,), jnp.int32),
            pltpu.VMEM((CHUNK, H), jnp.float32),
            pltpu.VMEM((CHUNK, H), jnp.float32),
            pltpu.VMEM((16,), jnp.int32),
            pltpu.VMEM((16, H), jnp.float32),
            pltpu.SemaphoreType.DMA,
            pltpu.SemaphoreType.DMA,
            pltpu.SemaphoreType.DMA,
            pltpu.SemaphoreType.DMA,
            pltpu.SemaphoreType.DMA,
        ],
    )
    return fn(ids, word_emb, ann_idx_pad, ann_table)


def _ln(y, g, b):
    mu = jnp.mean(y, axis=1, keepdims=True)
    d = y - mu
    var = jnp.mean(d * d, axis=1, keepdims=True)
    return d * lax.rsqrt(var + LN_EPS) * g + b


def _tc_ln_body(wraw_ref, pos_ref, ttid_ref, tt_ref, g_ref, b_ref,
                sent_ref, emb_ref, part_ref):
    i = pl.program_id(0)
    r = lax.rem(i, B) * NBLK_B + i // B
    tt0 = tt_ref[pl.ds(0, 1), :]
    tt1 = tt_ref[pl.ds(1, 1), :]
    tts = ttid_ref[pl.ds(r, 1), :]
    is1 = (jnp.transpose(tts, (1, 0)) == 1)
    x = wraw_ref[...] + pos_ref[...] + jnp.where(is1, tt1, tt0)
    sent_ref[...] = x
    bsum = jnp.sum(x, axis=0, keepdims=True)
    row0 = (lax.broadcasted_iota(jnp.int32, (8, 1), 0) == 0)
    part_ref[...] = jnp.where(row0, bsum, 0.0)
    emb_ref[...] = _ln(x, g_ref[...], b_ref[...])


def _tc_ln(wraw, pos_emb, tt_ids, tt_emb, gamma, beta):
    return pl.pallas_call(
        _tc_ln_body,
        grid=(NBLK,),
        in_specs=[
            pl.BlockSpec((BLK, H), lambda i: ((i % B) * NBLK_B + i // B, 0)),
            pl.BlockSpec((BLK, H), lambda i: (i // B, 0)),
            pl.BlockSpec((NBLK, BLK), lambda i: (0, 0)),
            pl.BlockSpec((2, H), lambda i: (0, 0)),
            pl.BlockSpec((1, H), lambda i: (0, 0)),
            pl.BlockSpec((1, H), lambda i: (0, 0)),
        ],
        out_specs=[
            pl.BlockSpec((BLK, H), lambda i: ((i % B) * NBLK_B + i // B, 0)),
            pl.BlockSpec((BLK, H), lambda i: ((i % B) * NBLK_B + i // B, 0)),
            pl.BlockSpec((8, H), lambda i: ((i % B) * NBLK_B + i // B, 0)),
        ],
        out_shape=[
            jax.ShapeDtypeStruct((N, H), jnp.float32),
            jax.ShapeDtypeStruct((N, H), jnp.float32),
            jax.ShapeDtypeStruct((NBLK * 8, H), jnp.float32),
        ],
    )(wraw, pos_emb, tt_ids, tt_emb, gamma, beta)


def _tc_patch_body(emb_in_ref, sent_ref, part_ref, ann_ref, sw_ref, aw_ref,
                   g_ref, b_ref, emb_ref, annout_ref, ann_emb_s):
    b = pl.program_id(0)

    @pl.when(b == 0)
    def _():
        p = part_ref[...]
        io = lax.broadcasted_iota(jnp.int32, (8, NBLK * 8), 0)
        ii = lax.broadcasted_iota(jnp.int32, (8, NBLK * 8), 1)
        sel = (ii // (8 * NBLK_B) == io).astype(jnp.float32)
        m = lax.dot_general(sel, p, (((1,), (0,)), ((), ())),
                            precision=lax.Precision.HIGHEST,
                            preferred_element_type=jnp.float32) * (1.0 / S)
        ann = ann_ref[...]
        u = lax.dot_general(m, sw_ref[...], (((1,), (1,)), ((), ())),
                            precision=lax.Precision.HIGHEST,
                            preferred_element_type=jnp.float32)
        v = lax.dot_general(ann, aw_ref[...], (((1,), (1,)), ((), ())),
                            precision=lax.Precision.HIGHEST,
                            preferred_element_type=jnp.float32)
        alpha = jnp.sum(u * v, axis=1, keepdims=True)
        ann_emb = alpha * ann
        ann_emb_s[...] = ann_emb
        annout_ref[...] = lax.slice(ann_emb, (0, 0), (B, H))

    ann_emb_b = ann_emb_s[pl.ds(b, 1), :]
    y0 = _ln(sent_ref[pl.ds(0, 1), :] + ann_emb_b, g_ref[...], b_ref[...])
    row0 = (lax.broadcasted_iota(jnp.int32, (8, 1), 0) == 0)
    emb_ref[...] = jnp.where(row0, y0, emb_in_ref[...])


def _tc_alpha_patch(emb0, sent, partials, ann_rows, sent_W, annotator_W,
                    gamma, beta):
    return pl.pallas_call(
        _tc_patch_body,
        grid=(B,),
        in_specs=[
            pl.BlockSpec((8, H), lambda b: (b * (S // 8), 0)),
            pl.BlockSpec((8, H), lambda b: (b * (S // 8), 0)),
            pl.BlockSpec((NBLK * 8, H), lambda b: (0, 0)),
            pl.BlockSpec((8, H), lambda b: (0, 0)),
            pl.BlockSpec((H, H), lambda b: (0, 0)),
            pl.BlockSpec((H, H), lambda b: (0, 0)),
            pl.BlockSpec((1, H), lambda b: (0, 0)),
            pl.BlockSpec((1, H), lambda b: (0, 0)),
        ],
        out_specs=[
            pl.BlockSpec((8, H), lambda b: (b * (S // 8), 0)),
            pl.BlockSpec((B, H), lambda b: (0, 0)),
        ],
        out_shape=[
            jax.ShapeDtypeStruct((N, H), jnp.float32),
            jax.ShapeDtypeStruct((B, H), jnp.float32),
        ],
        input_output_aliases={0: 0},
        scratch_shapes=[pltpu.VMEM((8, H), jnp.float32)],
    )(emb0, sent, partials, ann_rows, sent_W, annotator_W, gamma, beta)


def kernel(input_ids, token_type_ids, annotator_ids, word_emb, tt_emb,
           pos_emb, sent_W, annotator_W, ann_table, ln_gamma, ln_beta):
    ids = input_ids.reshape(-1).astype(jnp.int32)
    tt_ids = token_type_ids.reshape(NBLK, BLK).astype(jnp.int32)
    ann_idx_pad = jnp.tile(annotator_ids.astype(jnp.int32), 16 // B)

    wraw, ann_rows = _sc_gather(ids, ann_idx_pad, word_emb, ann_table)
    sent, emb0, partials = _tc_ln(wraw, pos_emb, tt_ids, tt_emb,
                                  ln_gamma.reshape(1, H),
                                  ln_beta.reshape(1, H))
    emb, ann_emb = _tc_alpha_patch(emb0, sent, partials, ann_rows, sent_W,
                                   annotator_W, ln_gamma.reshape(1, H),
                                   ln_beta.reshape(1, H))
    return (emb.reshape(B, S, H), ann_emb, sent.reshape(B, S, H))
